# baseline (speedup 1.0000x reference)
import functools

import jax
import jax.numpy as jnp
from jax import lax
from jax.experimental import pallas as pl
from jax.experimental.pallas import tpu as pltpu
from jax.experimental.pallas import tpu_sc as plsc

_LANES = 16


@functools.lru_cache(maxsize=None)
def _build_sc_kernel(a_dim: int):
    mesh = plsc.VectorSubcoreMesh(core_axis_name="c", subcore_axis_name="s",
                                  num_cores=1)
    info = plsc.get_sparse_core_info()
    num_cores, num_subcores = 1, info.num_subcores
    num_workers = num_cores * num_subcores
    chunk = a_dim // num_workers
    assert chunk % _LANES == 0 and chunk % 8 == 0

    @functools.partial(
        pl.kernel,
        mesh=mesh,
        out_type=jax.ShapeDtypeStruct((a_dim,), jnp.float32),
        scratch_types=[
            pltpu.VMEM((_LANES,), jnp.float32),
            pltpu.VMEM((chunk,), jnp.float32),
            pltpu.VMEM((chunk,), jnp.float32),
            pltpu.VMEM((chunk,), jnp.float32),
            pltpu.SemaphoreType.DMA,
        ],
    )
    def sc_kernel(cw_hbm, means_hbm, logsigs_hbm, eps_hbm, out_hbm,
                  cw_v, m_v, ls_v, e_v, sem):
        wid = lax.axis_index("s") * num_cores + lax.axis_index("c")
        base = wid * chunk
        c2 = pltpu.async_copy(logsigs_hbm.at[pl.ds(base, chunk)], ls_v, sem)
        c3 = pltpu.async_copy(eps_hbm.at[pl.ds(base, chunk)], e_v, sem)
        c1 = pltpu.async_copy(means_hbm.at[pl.ds(base, chunk)], m_v, sem)
        c0 = pltpu.async_copy(cw_hbm, cw_v.at[pl.ds(0, 1)], sem)
        c2.wait()
        c3.wait()
        for j in range(chunk // _LANES):
            sl = pl.ds(j * _LANES, _LANES)
            e_v[sl] = e_v[sl] * jnp.exp(ls_v[sl])
        c0.wait()
        c1.wait()
        cw = jnp.abs(jnp.full((_LANES,), cw_v[...][0], jnp.float32))
        w0 = cw / (cw + 1.0)
        half = chunk // 2
        for j in range(half // _LANES):
            sl = pl.ds(j * _LANES, _LANES)
            m_v[sl] = w0 * m_v[sl] + e_v[sl]
        o1 = pltpu.async_copy(m_v.at[pl.ds(0, half)],
                              out_hbm.at[pl.ds(base, half)], sem)
        for j in range(half // _LANES, chunk // _LANES):
            sl = pl.ds(j * _LANES, _LANES)
            m_v[sl] = w0 * m_v[sl] + e_v[sl]
        o2 = pltpu.async_copy(m_v.at[pl.ds(half, half)],
                              out_hbm.at[pl.ds(base + half, half)], sem)
        o1.wait()
        o2.wait()

    return sc_kernel


def kernel(s, cweights, means, logsigs, logtemp, eps):
    a_dim = s.shape[0]
    out = _build_sc_kernel(a_dim)(
        cweights, means.reshape(a_dim), logsigs, eps.reshape(a_dim))
    return out.reshape(1, a_dim)


# speedup vs baseline: 1.0098x; 1.0098x over previous
import functools

import jax
import jax.numpy as jnp
from jax import lax
from jax.experimental import pallas as pl
from jax.experimental.pallas import tpu as pltpu
from jax.experimental.pallas import tpu_sc as plsc

_LANES = 16


@functools.lru_cache(maxsize=None)
def _build_sc_kernel(a_dim: int):
    mesh = plsc.VectorSubcoreMesh(core_axis_name="c", subcore_axis_name="s",
                                  num_cores=1)
    info = plsc.get_sparse_core_info()
    num_cores, num_subcores = 1, info.num_subcores
    num_workers = num_cores * num_subcores
    chunk = a_dim // num_workers
    assert chunk % _LANES == 0 and chunk % 8 == 0

    @functools.partial(
        pl.kernel,
        mesh=mesh,
        out_type=jax.ShapeDtypeStruct((a_dim,), jnp.float32),
        scratch_types=[
            pltpu.VMEM((_LANES,), jnp.float32),
            pltpu.VMEM((chunk,), jnp.float32),
            pltpu.VMEM((chunk,), jnp.float32),
            pltpu.VMEM((chunk,), jnp.float32),
            pltpu.SemaphoreType.DMA,
        ],
    )
    def sc_kernel(cw_hbm, means_hbm, logsigs_hbm, eps_hbm, out_hbm,
                  cw_v, m_v, ls_v, e_v, sem):
        wid = lax.axis_index("s") * num_cores + lax.axis_index("c")
        base = wid * chunk
        c0 = pltpu.async_copy(cw_hbm, cw_v.at[pl.ds(0, 1)], sem)
        c1 = pltpu.async_copy(means_hbm.at[pl.ds(base, chunk)], m_v, sem)
        c2 = pltpu.async_copy(logsigs_hbm.at[pl.ds(base, chunk)], ls_v, sem)
        c3 = pltpu.async_copy(eps_hbm.at[pl.ds(base, chunk)], e_v, sem)
        c0.wait()
        c1.wait()
        c2.wait()
        c3.wait()
        cw = jnp.abs(jnp.full((_LANES,), cw_v[...][0], jnp.float32))
        w0 = cw / (cw + 1.0)
        for j in range(chunk // _LANES):
            sl = pl.ds(j * _LANES, _LANES)
            m_v[sl] = w0 * m_v[sl] + e_v[sl] * jnp.exp(ls_v[sl])
        pltpu.sync_copy(m_v, out_hbm.at[pl.ds(base, chunk)])

    return sc_kernel


def kernel(s, cweights, means, logsigs, logtemp, eps):
    a_dim = s.shape[0]
    out = _build_sc_kernel(a_dim)(
        cweights, means.reshape(a_dim), logsigs, eps.reshape(a_dim))
    return out.reshape(1, a_dim)
